# baseline (speedup 1.0000x reference)
import jax
import jax.numpy as jnp
from jax import lax
from jax.experimental import pallas as pl
from jax.experimental.pallas import tpu as pltpu
from jax.experimental.pallas import tpu_sc as plsc

R = 32
N = 4096
KSEL = 64
SL = 72
NB = 16
BLK = N // NB
LANES = NB * R
NEGC = -1e9
TOPBIT = -2**31
UNROLL = 4


def _lae(x, y):
    mx = jnp.maximum(x, y)
    return mx + jnp.log1p(jnp.exp(-jnp.abs(x - y)))


def _shift_dn(x, fill):
    return jnp.concatenate([jnp.full((1, x.shape[1]), fill, x.dtype), x[:-1, :]], axis=0)


def _shift_up(x, fill):
    return jnp.concatenate([x[1:, :], jnp.full((1, x.shape[1]), fill, x.dtype)], axis=0)


def _conv_normal(xtab, ttab):
    acc0 = jnp.full(xtab.shape, NEGC, xtab.dtype)

    def body(_, carry):
        acc, xsh, tsh = carry
        acc = _lae(acc, xsh[0:1, :] + tsh)
        return acc, _shift_up(xsh, NEGC), _shift_dn(tsh, NEGC)

    acc, _, _ = jax.lax.fori_loop(0, KSEL + 1, body, (acc0, xtab, ttab))
    return acc


def _conv_rev(xrev, ttab):
    acc0 = jnp.full(xrev.shape, NEGC, xrev.dtype)

    def body(_, carry):
        acc, xsh, tsh = carry
        acc = _lae(acc, xsh + tsh[0:1, :])
        return acc, _shift_up(xsh, NEGC), _shift_up(tsh, NEGC)

    acc, _, _ = jax.lax.fori_loop(0, KSEL + 1, body, (acc0, xrev, ttab))
    return acc


def _conv_rev2(xrev, yrev):
    acc0 = jnp.full(xrev.shape, NEGC, xrev.dtype)

    def body(_, carry):
        acc, xsh, ysh = carry
        acc = _lae(acc, xsh[SL - 1:SL, :] + ysh)
        return acc, _shift_dn(xsh, NEGC), _shift_up(ysh, NEGC)

    acc, _, _ = jax.lax.fori_loop(0, KSEL + 1, body, (acc0, xrev, yrev))
    return acc


def _empty_tab(lanes):
    o = jax.lax.broadcasted_iota(jnp.int32, (SL, lanes), 0)
    return jnp.where(o == 0, 0.0, NEGC).astype(jnp.float32)


def _empty_tab_rev(lanes):
    o = jax.lax.broadcasted_iota(jnp.int32, (SL, lanes), 0)
    return jnp.where(o == SL - 1, 0.0, NEGC).astype(jnp.float32)


def _sklansky(tabs, conv):
    n = len(tabs)
    c = list(tabs)
    dist = 1
    while dist < n:
        idx = [i for i in range(n) if (i // dist) % 2 == 1]
        xs = jnp.concatenate([c[(i // dist) * dist - 1] for i in idx], axis=1)
        ts = jnp.concatenate([c[i] for i in idx], axis=1)
        out = conv(xs, ts)
        w = tabs[0].shape[1]
        for j, i in enumerate(idx):
            c[i] = out[:, j * w:(j + 1) * w]
        dist *= 2
    return c


LSC = 16
NCHUNK = N // LSC


def _sc_topk_body(pert_hbm, bits_hbm, mask_hbm, prow_v, keys_v, mrow_v, cbuf_v, gbuf_v, ucand_v, bits_v):
    wid = lax.axis_index("s") * 2 + lax.axis_index("c")
    pltpu.sync_copy(pert_hbm.at[wid], prow_v)
    pltpu.sync_copy(bits_hbm, bits_v)
    top = jnp.int32(TOPBIT)
    zero16 = jnp.zeros((LSC,), jnp.int32)
    one16 = jnp.full((LSC,), 1, jnp.int32)
    m1_16 = jnp.full((LSC,), -1, jnp.int32)
    iota = lax.iota(jnp.int32, LSC)
    rots = [(iota + s) & (LSC - 1) for s in (1, 2, 4, 8)]

    def keys_body(i, c):
        for q in range(4):
            x = prow_v[pl.ds((i * 4 + q) * LSC, LSC)]
            ib = jax.lax.bitcast_convert_type(x, jnp.int32)
            keys_v[pl.ds((i * 4 + q) * LSC, LSC)] = jnp.where(ib < 0, ~ib ^ top, ib)
        return c

    lax.fori_loop(0, NCHUNK // 4, keys_body, 0)

    ucand_v[:] = zero16

    def bit_body(b, c):
        bitv = bits_v[pl.ds(b * LSC, LSC)]
        ucand = ucand_v[:]
        utest = ucand | bitv
        itest = utest ^ top
        cbuf_v[:] = zero16

        def count_body(i, c2):
            acc = zero16
            for q in range(8):
                ge = keys_v[pl.ds((i * 8 + q) * LSC, LSC)] >= itest
                acc = acc + jnp.where(ge, one16, zero16)
            cbuf_v[:] = cbuf_v[:] + acc
            return c2

        lax.fori_loop(0, NCHUNK // 8, count_body, 0)
        cnt = cbuf_v[:]
        tot = cnt[0]
        for j in range(1, LSC):
            tot = tot + cnt[j]
        t = jnp.where(tot >= KSEL, jnp.int32(-1), jnp.int32(0))
        ucand_v[:] = ucand | (bitv & t)
        return c

    lax.fori_loop(0, 32, bit_body, 0)
    ithresh = ucand_v[:] ^ top

    def mask_body(i, c):
        for q in range(4):
            ge = keys_v[pl.ds((i * 4 + q) * LSC, LSC)] >= ithresh
            mrow_v[pl.ds((i * 4 + q) * LSC, LSC)] = jnp.where(ge, 1.0, 0.0)
        return c

    lax.fori_loop(0, NCHUNK // 4, mask_body, 0)
    pltpu.sync_copy(mrow_v, mask_hbm.at[wid])


def _sc_topk(pert):
    import numpy as _np
    bits = jnp.asarray(_np.repeat(
        _np.array([-2**31] + [1 << (31 - r) for r in range(1, 32)],
                  dtype=_np.int32), LSC))
    mesh = plsc.VectorSubcoreMesh(core_axis_name="c", subcore_axis_name="s")
    f = pl.kernel(
        _sc_topk_body,
        mesh=mesh,
        out_type=jax.ShapeDtypeStruct((R, N), jnp.float32),
        scratch_types=[
            pltpu.VMEM((N,), jnp.float32),
            pltpu.VMEM((N,), jnp.int32),
            pltpu.VMEM((N,), jnp.float32),
            pltpu.VMEM((LSC,), jnp.int32),
            pltpu.VMEM((LSC,), jnp.int32),
            pltpu.VMEM((LSC,), jnp.int32),
            pltpu.VMEM((32 * LSC,), jnp.int32),
        ],
    )
    return f(pert, bits)


def _marg_body(a_ref, mask_ref, marg_ref, comb_ref, g_ref):
    def p1(tc, state):
        for q in range(UNROLL):
            col = a_ref[tc * UNROLL + q]
            state = _lae(state, _shift_dn(state, NEGC) + col)
        return state

    tfull = jax.lax.fori_loop(0, BLK // UNROLL, p1, _empty_tab(LANES))
    o = jax.lax.broadcasted_iota(jnp.int32, (SL, LANES), 0)
    tfull = jnp.where(o <= KSEL, tfull, NEGC)

    tblk = [tfull[:, b * R:(b + 1) * R] for b in range(NB)]

    cpre = _sklansky(tblk, _conv_normal)
    allt = jnp.concatenate(tblk, axis=1)
    allrev = _conv_rev(_empty_tab_rev(LANES), allt)
    trev = [allrev[:, b * R:(b + 1) * R] for b in range(NB)]
    csuf = _sklansky(trev[::-1], _conv_rev2)

    pinit = jnp.concatenate([_empty_tab(R)] + cpre[:NB - 1], axis=1)
    sinit = jnp.concatenate([csuf[NB - 2 - b] if b < NB - 1 else _empty_tab_rev(R)
                             for b in range(NB)], axis=1)
    fk32 = cpre[NB - 1][KSEL:KSEL + 1, :]
    fk = jnp.concatenate([fk32] * NB, axis=1)

    def p3b(ic, state):
        for q in range(UNROLL):
            t = BLK - 1 - (ic * UNROLL + q)
            g_ref[t] = state
            col = a_ref[t]
            state = _lae(state, _shift_up(state, NEGC) + col)
        return state

    jax.lax.fori_loop(0, BLK // UNROLL, p3b, sinit)

    def p3f(tc, state):
        for q in range(UNROLL):
            t = tc * UNROLL + q
            grev = g_ref[t]
            terms = state[0:KSEL, :] + (grev[SL - KSEL:SL, :] - fk)
            s = jnp.sum(jnp.exp(terms), axis=0, keepdims=True)
            col = a_ref[t]
            marg = jnp.exp(col) * s
            mk = mask_ref[t]
            marg_ref[t] = marg
            comb_ref[t] = (mk - marg) + marg
            state = _lae(state, _shift_dn(state, NEGC) + col)
        return state

    jax.lax.fori_loop(0, BLK // UNROLL, p3f, pinit)


def kernel(scores):
    bsz, nmax, ens = scores.shape
    flat = jnp.transpose(scores, (0, 2, 1)).reshape(bsz * ens, nmax)

    kk = jax.random.fold_in(jax.random.key(1), 0)
    u = jax.random.uniform(kk, flat.shape, minval=1e-20, maxval=1.0)
    pert = flat + (-jnp.log(-jnp.log(u)))

    mask = _sc_topk(pert)

    def steps(x):
        return x.reshape(R, NB, BLK).transpose(2, 1, 0).reshape(BLK, 1, LANES)

    def unsteps(x):
        return x.reshape(BLK, NB, R).transpose(2, 1, 0).reshape(R, N)

    marg_s, comb_s = pl.pallas_call(
        _marg_body,
        out_shape=(
            jax.ShapeDtypeStruct((BLK, 1, LANES), jnp.float32),
            jax.ShapeDtypeStruct((BLK, 1, LANES), jnp.float32),
        ),
        scratch_shapes=[pltpu.VMEM((BLK, SL, LANES), jnp.float32)],
        interpret=False,
    )(steps(flat), steps(mask))

    marg = unsteps(marg_s)
    comb = unsteps(comb_s)
    new_mask = comb.reshape(1, bsz, ens, nmax).transpose(0, 1, 3, 2)
    new_marg = marg.reshape(bsz, ens, nmax).transpose(0, 2, 1)
    return new_mask, new_marg


# speedup vs baseline: 1.2767x; 1.2767x over previous
import jax
import jax.numpy as jnp
from jax import lax
from jax.experimental import pallas as pl
from jax.experimental.pallas import tpu as pltpu
from jax.experimental.pallas import tpu_sc as plsc

R = 32
N = 4096
KSEL = 64
SL = 72
NB = 8
BLK = N // NB
LANES = NB * R
NEGC = -1e9
TOPBIT = -2**31
UNROLL = 8


def _lae(x, y):
    mx = jnp.maximum(x, y)
    return mx + jnp.log1p(jnp.exp(-jnp.abs(x - y)))


def _shift_dn(x, fill):
    return jnp.concatenate([jnp.full((1, x.shape[1]), fill, x.dtype), x[:-1, :]], axis=0)


def _shift_up(x, fill):
    return jnp.concatenate([x[1:, :], jnp.full((1, x.shape[1]), fill, x.dtype)], axis=0)


def _conv_normal(xtab, ttab):
    acc0 = jnp.full(xtab.shape, NEGC, xtab.dtype)

    def body(_, carry):
        acc, xsh, tsh = carry
        acc = _lae(acc, xsh[0:1, :] + tsh)
        return acc, _shift_up(xsh, NEGC), _shift_dn(tsh, NEGC)

    acc, _, _ = jax.lax.fori_loop(0, KSEL + 1, body, (acc0, xtab, ttab))
    return acc


def _conv_rev(xrev, ttab):
    acc0 = jnp.full(xrev.shape, NEGC, xrev.dtype)

    def body(_, carry):
        acc, xsh, tsh = carry
        acc = _lae(acc, xsh + tsh[0:1, :])
        return acc, _shift_up(xsh, NEGC), _shift_up(tsh, NEGC)

    acc, _, _ = jax.lax.fori_loop(0, KSEL + 1, body, (acc0, xrev, ttab))
    return acc


def _conv_rev2(xrev, yrev):
    acc0 = jnp.full(xrev.shape, NEGC, xrev.dtype)

    def body(_, carry):
        acc, xsh, ysh = carry
        acc = _lae(acc, xsh[SL - 1:SL, :] + ysh)
        return acc, _shift_dn(xsh, NEGC), _shift_up(ysh, NEGC)

    acc, _, _ = jax.lax.fori_loop(0, KSEL + 1, body, (acc0, xrev, yrev))
    return acc


def _empty_tab(lanes):
    o = jax.lax.broadcasted_iota(jnp.int32, (SL, lanes), 0)
    return jnp.where(o == 0, 0.0, NEGC).astype(jnp.float32)


def _empty_tab_rev(lanes):
    o = jax.lax.broadcasted_iota(jnp.int32, (SL, lanes), 0)
    return jnp.where(o == SL - 1, 0.0, NEGC).astype(jnp.float32)


def _sklansky(tabs, conv):
    n = len(tabs)
    c = list(tabs)
    dist = 1
    while dist < n:
        idx = [i for i in range(n) if (i // dist) % 2 == 1]
        xs = jnp.concatenate([c[(i // dist) * dist - 1] for i in idx], axis=1)
        ts = jnp.concatenate([c[i] for i in idx], axis=1)
        out = conv(xs, ts)
        w = tabs[0].shape[1]
        for j, i in enumerate(idx):
            c[i] = out[:, j * w:(j + 1) * w]
        dist *= 2
    return c


LSC = 16
NCHUNK = N // LSC


def _sc_topk_body(pert_hbm, bits_hbm, mask_hbm, prow_v, keys_v, mrow_v, cbuf_v, gbuf_v, ucand_v, bits_v):
    wid = lax.axis_index("s") * 2 + lax.axis_index("c")
    pltpu.sync_copy(pert_hbm.at[wid], prow_v)
    pltpu.sync_copy(bits_hbm, bits_v)
    top = jnp.int32(TOPBIT)
    zero16 = jnp.zeros((LSC,), jnp.int32)
    one16 = jnp.full((LSC,), 1, jnp.int32)
    m1_16 = jnp.full((LSC,), -1, jnp.int32)
    iota = lax.iota(jnp.int32, LSC)
    rots = [(iota + s) & (LSC - 1) for s in (1, 2, 4, 8)]

    def keys_body(i, c):
        for q in range(4):
            x = prow_v[pl.ds((i * 4 + q) * LSC, LSC)]
            ib = jax.lax.bitcast_convert_type(x, jnp.int32)
            keys_v[pl.ds((i * 4 + q) * LSC, LSC)] = jnp.where(ib < 0, ~ib ^ top, ib)
        return c

    lax.fori_loop(0, NCHUNK // 4, keys_body, 0)

    ucand_v[:] = zero16

    def bit_body(b, c):
        bitv = bits_v[pl.ds(b * LSC, LSC)]
        ucand = ucand_v[:]
        utest = ucand | bitv
        itest = utest ^ top
        cbuf_v[:] = zero16

        def count_body(i, c2):
            acc = zero16
            for q in range(8):
                ge = keys_v[pl.ds((i * 8 + q) * LSC, LSC)] >= itest
                acc = acc + jnp.where(ge, one16, zero16)
            cbuf_v[:] = cbuf_v[:] + acc
            return c2

        lax.fori_loop(0, NCHUNK // 8, count_body, 0)
        cnt = cbuf_v[:]
        tot = cnt[0]
        for j in range(1, LSC):
            tot = tot + cnt[j]
        t = jnp.where(tot >= KSEL, jnp.int32(-1), jnp.int32(0))
        ucand_v[:] = ucand | (bitv & t)
        return c

    lax.fori_loop(0, 32, bit_body, 0)
    ithresh = ucand_v[:] ^ top

    def mask_body(i, c):
        for q in range(4):
            ge = keys_v[pl.ds((i * 4 + q) * LSC, LSC)] >= ithresh
            mrow_v[pl.ds((i * 4 + q) * LSC, LSC)] = jnp.where(ge, 1.0, 0.0)
        return c

    lax.fori_loop(0, NCHUNK // 4, mask_body, 0)
    pltpu.sync_copy(mrow_v, mask_hbm.at[wid])


def _sc_topk(pert):
    import numpy as _np
    bits = jnp.asarray(_np.repeat(
        _np.array([-2**31] + [1 << (31 - r) for r in range(1, 32)],
                  dtype=_np.int32), LSC))
    mesh = plsc.VectorSubcoreMesh(core_axis_name="c", subcore_axis_name="s")
    f = pl.kernel(
        _sc_topk_body,
        mesh=mesh,
        out_type=jax.ShapeDtypeStruct((R, N), jnp.float32),
        scratch_types=[
            pltpu.VMEM((N,), jnp.float32),
            pltpu.VMEM((N,), jnp.int32),
            pltpu.VMEM((N,), jnp.float32),
            pltpu.VMEM((LSC,), jnp.int32),
            pltpu.VMEM((LSC,), jnp.int32),
            pltpu.VMEM((LSC,), jnp.int32),
            pltpu.VMEM((32 * LSC,), jnp.int32),
        ],
    )
    return f(pert, bits)


def _marg_body(a_ref, mask_ref, marg_ref, comb_ref, g_ref):
    def p1(tc, state):
        for q in range(UNROLL):
            col = a_ref[tc * UNROLL + q]
            state = _lae(state, _shift_dn(state, NEGC) + col)
        return state

    tfull = jax.lax.fori_loop(0, BLK // UNROLL, p1, _empty_tab(LANES))
    o = jax.lax.broadcasted_iota(jnp.int32, (SL, LANES), 0)
    tfull = jnp.where(o <= KSEL, tfull, NEGC)

    tblk = [tfull[:, b * R:(b + 1) * R] for b in range(NB)]

    cpre = _sklansky(tblk, _conv_normal)
    allt = jnp.concatenate(tblk, axis=1)
    allrev = _conv_rev(_empty_tab_rev(LANES), allt)
    trev = [allrev[:, b * R:(b + 1) * R] for b in range(NB)]
    csuf = _sklansky(trev[::-1], _conv_rev2)

    pinit = jnp.concatenate([_empty_tab(R)] + cpre[:NB - 1], axis=1)
    sinit = jnp.concatenate([csuf[NB - 2 - b] if b < NB - 1 else _empty_tab_rev(R)
                             for b in range(NB)], axis=1)
    fk32 = cpre[NB - 1][KSEL:KSEL + 1, :]
    fk = jnp.concatenate([fk32] * NB, axis=1)

    def p3b(ic, state):
        for q in range(UNROLL):
            t = BLK - 1 - (ic * UNROLL + q)
            g_ref[t] = state[SL - KSEL:SL, :]
            col = a_ref[t]
            state = _lae(state, _shift_up(state, NEGC) + col)
        return state

    jax.lax.fori_loop(0, BLK // UNROLL, p3b, sinit)

    def p3f(tc, state):
        for q in range(UNROLL):
            t = tc * UNROLL + q
            grev = g_ref[t]
            terms = state[0:KSEL, :] + (grev - fk)
            s = jnp.sum(jnp.exp(terms), axis=0, keepdims=True)
            col = a_ref[t]
            marg = jnp.exp(col) * s
            mk = mask_ref[t]
            marg_ref[t] = marg
            comb_ref[t] = (mk - marg) + marg
            state = _lae(state, _shift_dn(state, NEGC) + col)
        return state

    jax.lax.fori_loop(0, BLK // UNROLL, p3f, pinit)


def kernel(scores):
    bsz, nmax, ens = scores.shape
    flat = jnp.transpose(scores, (0, 2, 1)).reshape(bsz * ens, nmax)

    kk = jax.random.fold_in(jax.random.key(1), 0)
    u = jax.random.uniform(kk, flat.shape, minval=1e-20, maxval=1.0)
    pert = flat + (-jnp.log(-jnp.log(u)))

    mask = _sc_topk(pert)

    def steps(x):
        return x.reshape(R, NB, BLK).transpose(2, 1, 0).reshape(BLK, 1, LANES)

    def unsteps(x):
        return x.reshape(BLK, NB, R).transpose(2, 1, 0).reshape(R, N)

    marg_s, comb_s = pl.pallas_call(
        _marg_body,
        out_shape=(
            jax.ShapeDtypeStruct((BLK, 1, LANES), jnp.float32),
            jax.ShapeDtypeStruct((BLK, 1, LANES), jnp.float32),
        ),
        scratch_shapes=[pltpu.VMEM((BLK, KSEL, LANES), jnp.float32)],
        interpret=False,
    )(steps(flat), steps(mask))

    marg = unsteps(marg_s)
    comb = unsteps(comb_s)
    new_mask = comb.reshape(1, bsz, ens, nmax).transpose(0, 1, 3, 2)
    new_marg = marg.reshape(bsz, ens, nmax).transpose(0, 2, 1)
    return new_mask, new_marg


# speedup vs baseline: 1.4523x; 1.1376x over previous
import jax
import jax.numpy as jnp
from jax import lax
from jax.experimental import pallas as pl
from jax.experimental.pallas import tpu as pltpu
from jax.experimental.pallas import tpu_sc as plsc

R = 32
N = 4096
KSEL = 64
SL = 72
NB = 8
BLK = N // NB
LANES = NB * R
NEGC = -1e9
TOPBIT = -2**31
UNROLL = 8


LOG2E = 1.4426950408889634


def _lae(x, y):
    mx = jnp.maximum(x, y)
    return mx + jnp.log2(1.0 + jnp.exp2(-jnp.abs(x - y)))


def _shift_dn(x, fill):
    return jnp.concatenate([jnp.full((1, x.shape[1]), fill, x.dtype), x[:-1, :]], axis=0)


def _shift_up(x, fill):
    return jnp.concatenate([x[1:, :], jnp.full((1, x.shape[1]), fill, x.dtype)], axis=0)


def _conv_normal(xtab, ttab):
    acc0 = jnp.full(xtab.shape, NEGC, xtab.dtype)

    def body(_, carry):
        acc, xsh, tsh = carry
        acc = _lae(acc, xsh[0:1, :] + tsh)
        return acc, _shift_up(xsh, NEGC), _shift_dn(tsh, NEGC)

    acc, _, _ = jax.lax.fori_loop(0, KSEL + 1, body, (acc0, xtab, ttab))
    return acc


def _conv_rev(xrev, ttab):
    acc0 = jnp.full(xrev.shape, NEGC, xrev.dtype)

    def body(_, carry):
        acc, xsh, tsh = carry
        acc = _lae(acc, xsh + tsh[0:1, :])
        return acc, _shift_up(xsh, NEGC), _shift_up(tsh, NEGC)

    acc, _, _ = jax.lax.fori_loop(0, KSEL + 1, body, (acc0, xrev, ttab))
    return acc


def _conv_rev2(xrev, yrev):
    acc0 = jnp.full(xrev.shape, NEGC, xrev.dtype)

    def body(_, carry):
        acc, xsh, ysh = carry
        acc = _lae(acc, xsh[SL - 1:SL, :] + ysh)
        return acc, _shift_dn(xsh, NEGC), _shift_up(ysh, NEGC)

    acc, _, _ = jax.lax.fori_loop(0, KSEL + 1, body, (acc0, xrev, yrev))
    return acc


def _empty_tab(lanes):
    o = jax.lax.broadcasted_iota(jnp.int32, (SL, lanes), 0)
    return jnp.where(o == 0, 0.0, NEGC).astype(jnp.float32)


def _empty_tab_rev(lanes):
    o = jax.lax.broadcasted_iota(jnp.int32, (SL, lanes), 0)
    return jnp.where(o == SL - 1, 0.0, NEGC).astype(jnp.float32)


def _sklansky(tabs, conv):
    n = len(tabs)
    c = list(tabs)
    dist = 1
    while dist < n:
        idx = [i for i in range(n) if (i // dist) % 2 == 1]
        xs = jnp.concatenate([c[(i // dist) * dist - 1] for i in idx], axis=1)
        ts = jnp.concatenate([c[i] for i in idx], axis=1)
        out = conv(xs, ts)
        w = tabs[0].shape[1]
        for j, i in enumerate(idx):
            c[i] = out[:, j * w:(j + 1) * w]
        dist *= 2
    return c


LSC = 16
NCHUNK = N // LSC


def _sc_topk_body(pert_hbm, bits_hbm, mask_hbm, prow_v, keys_v, mrow_v, cbuf_v, gbuf_v, ucand_v, bits_v):
    wid = lax.axis_index("s") * 2 + lax.axis_index("c")
    pltpu.sync_copy(pert_hbm.at[wid], prow_v)
    pltpu.sync_copy(bits_hbm, bits_v)
    top = jnp.int32(TOPBIT)
    zero16 = jnp.zeros((LSC,), jnp.int32)
    one16 = jnp.full((LSC,), 1, jnp.int32)
    m1_16 = jnp.full((LSC,), -1, jnp.int32)
    iota = lax.iota(jnp.int32, LSC)
    rots = [(iota + s) & (LSC - 1) for s in (1, 2, 4, 8)]

    def keys_body(i, c):
        for q in range(4):
            x = prow_v[pl.ds((i * 4 + q) * LSC, LSC)]
            ib = jax.lax.bitcast_convert_type(x, jnp.int32)
            keys_v[pl.ds((i * 4 + q) * LSC, LSC)] = jnp.where(ib < 0, ~ib ^ top, ib)
        return c

    lax.fori_loop(0, NCHUNK // 4, keys_body, 0)

    ucand_v[:] = zero16

    def bit_body(b, c):
        bitv = bits_v[pl.ds(b * LSC, LSC)]
        ucand = ucand_v[:]
        utest = ucand | bitv
        itest = utest ^ top
        cbuf_v[:] = zero16

        def count_body(i, c2):
            acc = zero16
            for q in range(8):
                ge = keys_v[pl.ds((i * 8 + q) * LSC, LSC)] >= itest
                acc = acc + jnp.where(ge, one16, zero16)
            cbuf_v[:] = cbuf_v[:] + acc
            return c2

        lax.fori_loop(0, NCHUNK // 8, count_body, 0)
        cnt = cbuf_v[:]
        tot = cnt[0]
        for j in range(1, LSC):
            tot = tot + cnt[j]
        t = jnp.where(tot >= KSEL, jnp.int32(-1), jnp.int32(0))
        ucand_v[:] = ucand | (bitv & t)
        return c

    lax.fori_loop(0, 32, bit_body, 0)
    ithresh = ucand_v[:] ^ top

    def mask_body(i, c):
        for q in range(4):
            ge = keys_v[pl.ds((i * 4 + q) * LSC, LSC)] >= ithresh
            mrow_v[pl.ds((i * 4 + q) * LSC, LSC)] = jnp.where(ge, 1.0, 0.0)
        return c

    lax.fori_loop(0, NCHUNK // 4, mask_body, 0)
    pltpu.sync_copy(mrow_v, mask_hbm.at[wid])


def _sc_topk(pert):
    import numpy as _np
    bits = jnp.asarray(_np.repeat(
        _np.array([-2**31] + [1 << (31 - r) for r in range(1, 32)],
                  dtype=_np.int32), LSC))
    mesh = plsc.VectorSubcoreMesh(core_axis_name="c", subcore_axis_name="s")
    f = pl.kernel(
        _sc_topk_body,
        mesh=mesh,
        out_type=jax.ShapeDtypeStruct((R, N), jnp.float32),
        scratch_types=[
            pltpu.VMEM((N,), jnp.float32),
            pltpu.VMEM((N,), jnp.int32),
            pltpu.VMEM((N,), jnp.float32),
            pltpu.VMEM((LSC,), jnp.int32),
            pltpu.VMEM((LSC,), jnp.int32),
            pltpu.VMEM((LSC,), jnp.int32),
            pltpu.VMEM((32 * LSC,), jnp.int32),
        ],
    )
    return f(pert, bits)


def _marg_body(a_ref, mask_ref, marg_ref, comb_ref, g_ref):
    def p1(tc, state):
        for q in range(UNROLL):
            col = a_ref[tc * UNROLL + q] * LOG2E
            state = _lae(state, _shift_dn(state, NEGC) + col)
        return state

    tfull = jax.lax.fori_loop(0, BLK // UNROLL, p1, _empty_tab(LANES))
    o = jax.lax.broadcasted_iota(jnp.int32, (SL, LANES), 0)
    tfull = jnp.where(o <= KSEL, tfull, NEGC)

    tblk = [tfull[:, b * R:(b + 1) * R] for b in range(NB)]

    cpre = _sklansky(tblk, _conv_normal)
    allt = jnp.concatenate(tblk, axis=1)
    allrev = _conv_rev(_empty_tab_rev(LANES), allt)
    trev = [allrev[:, b * R:(b + 1) * R] for b in range(NB)]
    csuf = _sklansky(trev[::-1], _conv_rev2)

    pinit = jnp.concatenate([_empty_tab(R)] + cpre[:NB - 1], axis=1)
    sinit = jnp.concatenate([csuf[NB - 2 - b] if b < NB - 1 else _empty_tab_rev(R)
                             for b in range(NB)], axis=1)
    fk32 = cpre[NB - 1][KSEL:KSEL + 1, :]
    fk = jnp.concatenate([fk32] * NB, axis=1)

    def p3b(ic, state):
        for q in range(UNROLL):
            t = BLK - 1 - (ic * UNROLL + q)
            g_ref[t] = state[SL - KSEL:SL, :]
            col = a_ref[t] * LOG2E
            state = _lae(state, _shift_up(state, NEGC) + col)
        return state

    jax.lax.fori_loop(0, BLK // UNROLL, p3b, sinit)

    def p3f(tc, state):
        for q in range(UNROLL):
            t = tc * UNROLL + q
            grev = g_ref[t]
            terms = state[0:KSEL, :] + (grev - fk)
            s = jnp.sum(jnp.exp2(terms), axis=0, keepdims=True)
            col = a_ref[t] * LOG2E
            marg = jnp.exp2(col) * s
            mk = mask_ref[t]
            marg_ref[t] = marg
            comb_ref[t] = (mk - marg) + marg
            state = _lae(state, _shift_dn(state, NEGC) + col)
        return state

    jax.lax.fori_loop(0, BLK // UNROLL, p3f, pinit)


def kernel(scores):
    bsz, nmax, ens = scores.shape
    flat = jnp.transpose(scores, (0, 2, 1)).reshape(bsz * ens, nmax)

    kk = jax.random.fold_in(jax.random.key(1), 0)
    u = jax.random.uniform(kk, flat.shape, minval=1e-20, maxval=1.0)
    pert = flat + (-jnp.log(-jnp.log(u)))

    mask = _sc_topk(pert)

    def steps(x):
        return x.reshape(R, NB, BLK).transpose(2, 1, 0).reshape(BLK, 1, LANES)

    def unsteps(x):
        return x.reshape(BLK, NB, R).transpose(2, 1, 0).reshape(R, N)

    marg_s, comb_s = pl.pallas_call(
        _marg_body,
        out_shape=(
            jax.ShapeDtypeStruct((BLK, 1, LANES), jnp.float32),
            jax.ShapeDtypeStruct((BLK, 1, LANES), jnp.float32),
        ),
        scratch_shapes=[pltpu.VMEM((BLK, KSEL, LANES), jnp.float32)],
        interpret=False,
    )(steps(flat), steps(mask))

    marg = unsteps(marg_s)
    comb = unsteps(comb_s)
    new_mask = comb.reshape(1, bsz, ens, nmax).transpose(0, 1, 3, 2)
    new_marg = marg.reshape(bsz, ens, nmax).transpose(0, 2, 1)
    return new_mask, new_marg
